# baseline (speedup 1.0000x reference)
import functools

import jax
import jax.numpy as jnp
from jax import lax
from jax.experimental import pallas as pl
from jax.experimental.pallas import tpu as pltpu
from jax.experimental.pallas import tpu_sc as plsc

N_NODES_C = 10000
N_REL_C = 3
N_EDGES_C = 320000
D_C = 128
NPAD = 10240
ROW_BLK = 1024
NW = 32
RPW = NPAD // NW
CHUNK = 3200
NCHUNK = N_EDGES_C // CHUNK
G = 16
K = 16
NEG_INF = float("-inf")
NKS = D_C // 16


def _matmul_body(x_ref, w_ref, b_ref, y_ref):
    xb = x_ref[...]
    w = w_ref[0]
    acc = lax.dot_general(xb, w, (((1,), (1,)), ((), ())),
                          preferred_element_type=jnp.float32)
    y_ref[0] = acc + b_ref[0]


def _linear_all(x_pad, w_all, b_all):
    grid = (4, NPAD // ROW_BLK)
    return pl.pallas_call(
        _matmul_body,
        grid=grid,
        in_specs=[
            pl.BlockSpec((ROW_BLK, D_C), lambda j, i: (i, 0)),
            pl.BlockSpec((1, D_C, D_C), lambda j, i: (j, 0, 0)),
            pl.BlockSpec((1, 1, D_C), lambda j, i: (j, 0, 0)),
        ],
        out_specs=pl.BlockSpec((1, ROW_BLK, D_C), lambda j, i: (j, i, 0)),
        out_shape=jax.ShapeDtypeStruct((4, NPAD, D_C), jnp.float32),
    )(x_pad, w_all, b_all)


def _sum_body(y0_ref, a_ref, o_ref):
    o_ref[...] = y0_ref[...] + a_ref[0] + a_ref[1] + a_ref[2]


def _final_sum(y0, aggs):
    grid = (NPAD // ROW_BLK,)
    return pl.pallas_call(
        _sum_body,
        grid=grid,
        in_specs=[
            pl.BlockSpec((ROW_BLK, D_C), lambda i: (i, 0)),
            pl.BlockSpec((N_REL_C, ROW_BLK, D_C), lambda i: (0, i, 0)),
        ],
        out_specs=pl.BlockSpec((ROW_BLK, D_C), lambda i: (i, 0)),
        out_shape=jax.ShapeDtypeStruct((NPAD, D_C), jnp.float32),
    )(y0, aggs)


def _make_sc_aggregate():
    mesh = plsc.VectorSubcoreMesh(core_axis_name="c", subcore_axis_name="s",
                                  num_cores=2)

    @functools.partial(
        pl.kernel,
        out_type=jax.ShapeDtypeStruct((N_REL_C, NPAD, D_C), jnp.float32),
        mesh=mesh,
        compiler_params=pltpu.CompilerParams(needs_layout_passes=False),
        scratch_types=[
            pltpu.VMEM((CHUNK,), jnp.int32),
            pltpu.VMEM((CHUNK,), jnp.int32),
            pltpu.VMEM((CHUNK,), jnp.int32),
            pltpu.VMEM((CHUNK,), jnp.int32),
            pltpu.VMEM((CHUNK + G,), jnp.int32),
            pltpu.VMEM((CHUNK + G,), jnp.int32),
            pltpu.VMEM((2 * K * G, D_C), jnp.float32),
            pltpu.VMEM((RPW + 1, D_C), jnp.float32),
            pltpu.SemaphoreType.DMA,
            pltpu.SemaphoreType.DMA,
            pltpu.SemaphoreType.DMA,
            pltpu.SemaphoreType.DMA,
        ],
    )
    def sc_aggregate(h_hbm, ei_hbm, agg_hbm,
                     srcA, dstA, srcB, dstB, slist, llist,
                     ring, acc, semA, semB, semG0, semG1):
        cid = lax.axis_index("c")
        sid = lax.axis_index("s")
        wid = sid * 2 + cid
        lo = wid * RPW
        hi = lo + RPW

        ninf16 = jnp.full((16,), NEG_INF, jnp.float32)
        zero16 = jnp.zeros((16,), jnp.int32)
        trash16 = jnp.full((16,), RPW, jnp.int32)

        def issue_chunk(c, sbuf, dbuf, sem, soff, doff):
            pltpu.async_copy(ei_hbm.at[pl.ds(soff + c * CHUNK, CHUNK)],
                             sbuf, sem)
            pltpu.async_copy(ei_hbm.at[pl.ds(doff + c * CHUNK, CHUNK)],
                             dbuf, sem)

        def wait_chunk(sbuf, dbuf, sem):
            pltpu.make_async_copy(ei_hbm.at[pl.ds(0, CHUNK)], sbuf,
                                  sem).wait()
            pltpu.make_async_copy(ei_hbm.at[pl.ds(0, CHUNK)], dbuf,
                                  sem).wait()

        def rel_body(r, _):
            soff = (2 * r) * N_EDGES_C
            doff = soff + N_EDGES_C
            goff = r * NPAD

            def init_body(i, _):
                for k in range(NKS):
                    acc[i, pl.ds(k * 16, 16)] = ninf16
                return 0
            lax.fori_loop(0, RPW, init_body, 0)

            issue_chunk(0, srcA, dstA, semA, soff, doff)

            def process(c, sbuf, dbuf, sem, nsbuf, ndbuf, nsem):
                wait_chunk(sbuf, dbuf, sem)

                @pl.when(c + 1 < NCHUNK)
                def _():
                    issue_chunk(c + 1, nsbuf, ndbuf, nsem, soff, doff)

                def fbody(i, cnt):
                    dv0 = dbuf[pl.ds(i * 32, 16)]
                    dv1 = dbuf[pl.ds(i * 32 + 16, 16)]
                    m0 = (dv0 >= lo) & (dv0 < hi)
                    m1 = (dv1 >= lo) & (dv1 < hi)
                    sv0 = sbuf[pl.ds(i * 32, 16)] + goff
                    sv1 = sbuf[pl.ds(i * 32 + 16, 16)] + goff
                    plsc.store_compressed(llist.at[pl.ds(cnt, 16)],
                                          dv0 - lo, mask=m0)
                    plsc.store_compressed(slist.at[pl.ds(cnt, 16)], sv0,
                                          mask=m0)
                    pc0 = plsc.all_reduce_population_count(m0)
                    cnt1 = cnt + pc0[0]
                    plsc.store_compressed(llist.at[pl.ds(cnt1, 16)],
                                          dv1 - lo, mask=m1)
                    plsc.store_compressed(slist.at[pl.ds(cnt1, 16)], sv1,
                                          mask=m1)
                    pc1 = plsc.all_reduce_population_count(m1)
                    return cnt1 + pc1[0]

                n = lax.fori_loop(0, CHUNK // 32, fbody, jnp.int32(0))

                for t in range(G // 16):
                    slist[pl.ds(n + t * 16, 16)] = zero16
                    llist[pl.ds(n + t * 16, 16)] = trash16
                nblk = (n + (G - 1)) // G
                nsb = (nblk + (K - 1)) // K

                def fire_sb(s):
                    m = jnp.minimum(K, nblk - s * K)
                    sem_sel = lax.rem(s, 2)
                    base = sem_sel * (K * G)

                    def fire(b, _):
                        blk = s * K + b
                        dst = ring.at[pl.ds(base + b * G, G)]

                        @pl.when(sem_sel == 0)
                        def _():
                            pltpu.async_copy(
                                h_hbm.at[slist.at[pl.ds(blk * G, G)]],
                                dst, semG0)

                        @pl.when(sem_sel == 1)
                        def _():
                            pltpu.async_copy(
                                h_hbm.at[slist.at[pl.ds(blk * G, G)]],
                                dst, semG1)
                        return 0

                    lax.fori_loop(0, m, fire, 0)

                def sb_body(s, _):
                    @pl.when(s + 1 < nsb)
                    def _():
                        fire_sb(s + 1)

                    m = jnp.minimum(K, nblk - s * K)
                    sem_sel = lax.rem(s, 2)
                    base = sem_sel * (K * G)

                    def drain(b, _):
                        @pl.when(sem_sel == 0)
                        def _():
                            pltpu.make_async_copy(
                                h_hbm.at[slist.at[pl.ds(0, G)]],
                                ring.at[pl.ds(0, G)], semG0).wait()

                        @pl.when(sem_sel == 1)
                        def _():
                            pltpu.make_async_copy(
                                h_hbm.at[slist.at[pl.ds(0, G)]],
                                ring.at[pl.ds(0, G)], semG1).wait()
                        return 0

                    lax.fori_loop(0, m, drain, 0)

                    def mblk(b, _):
                        blk = s * K + b
                        roff = base + b * G
                        for g in range(G // 16):
                            lvec = llist[pl.ds(blk * G + g * 16, 16)]
                            for e in range(16):
                                dloc = lvec[e]
                                ri = roff + g * 16 + e
                                rv = [ring[ri, pl.ds(k * 16, 16)]
                                      for k in range(NKS)]
                                av = [acc[dloc, pl.ds(k * 16, 16)]
                                      for k in range(NKS)]
                                for k in range(NKS):
                                    acc[dloc, pl.ds(k * 16, 16)] = (
                                        jnp.maximum(av[k], rv[k]))
                        return 0

                    lax.fori_loop(0, m, mblk, 0)
                    return 0

                @pl.when(nsb > 0)
                def _():
                    fire_sb(0)
                lax.fori_loop(0, nsb, sb_body, 0)

            def pair_body(cc, _):
                c0 = 2 * cc
                process(c0, srcA, dstA, semA, srcB, dstB, semB)
                process(c0 + 1, srcB, dstB, semB, srcA, dstA, semA)
                return 0

            lax.fori_loop(0, NCHUNK // 2, pair_body, 0)

            def finish(i, _):
                for k in range(NKS):
                    sl = pl.ds(k * 16, 16)
                    a = acc[i, sl]
                    acc[i, sl] = jnp.where(a == NEG_INF, jnp.float32(0.0), a)
                return 0
            lax.fori_loop(0, RPW, finish, 0)

            pltpu.sync_copy(acc.at[pl.ds(0, RPW)],
                            agg_hbm.at[r, pl.ds(lo, RPW)])
            return 0

        lax.fori_loop(0, N_REL_C, rel_body, 0)

    return sc_aggregate


_sc_aggregate = None


def kernel(x, edge_indices_list, W_root, b_root, W_rel):
    global _sc_aggregate
    if _sc_aggregate is None:
        _sc_aggregate = _make_sc_aggregate()
    x_pad = jnp.pad(x, ((0, NPAD - N_NODES_C), (0, 0)))
    w_all = jnp.concatenate([W_root[None], W_rel], axis=0)
    b_all = jnp.concatenate(
        [b_root[None], jnp.zeros((3, D_C), jnp.float32)], 0
    ).reshape(4, 1, D_C)
    y = _linear_all(x_pad, w_all, b_all)
    h = y[1:].reshape(3 * NPAD, D_C)
    ei = edge_indices_list.astype(jnp.int32).reshape(-1)
    aggs = _sc_aggregate(h, ei)
    out_pad = _final_sum(y[0], aggs)
    return out_pad[:N_NODES_C]


# speedup vs baseline: 1.7320x; 1.7320x over previous
import functools

import jax
import jax.numpy as jnp
from jax import lax
from jax.experimental import pallas as pl
from jax.experimental.pallas import tpu as pltpu
from jax.experimental.pallas import tpu_sc as plsc

N_NODES_C = 10000
N_REL_C = 3
N_EDGES_C = 320000
D_C = 128
NPAD = 10240
ROW_BLK = 1024
NW = 32
RPW = NPAD // NW
CHUNK = 6400
NCHUNK = N_EDGES_C // CHUNK
G = 16
K = 4
NEG_INF = float("-inf")
NKS = D_C // 16


def _matmul_body(x_ref, w_ref, b_ref, y_ref):
    xb = x_ref[...]
    w = w_ref[0]
    acc = lax.dot_general(xb, w, (((1,), (1,)), ((), ())),
                          preferred_element_type=jnp.float32)
    y_ref[0] = acc + b_ref[0]


def _linear_all(x_pad, w_all, b_all):
    grid = (4, NPAD // ROW_BLK)
    return pl.pallas_call(
        _matmul_body,
        grid=grid,
        in_specs=[
            pl.BlockSpec((ROW_BLK, D_C), lambda j, i: (i, 0)),
            pl.BlockSpec((1, D_C, D_C), lambda j, i: (j, 0, 0)),
            pl.BlockSpec((1, 1, D_C), lambda j, i: (j, 0, 0)),
        ],
        out_specs=pl.BlockSpec((1, ROW_BLK, D_C), lambda j, i: (j, i, 0)),
        out_shape=jax.ShapeDtypeStruct((4, NPAD, D_C), jnp.float32),
    )(x_pad, w_all, b_all)


def _sum_body(y0_ref, a_ref, o_ref):
    o_ref[...] = y0_ref[...] + a_ref[0] + a_ref[1] + a_ref[2]


def _final_sum(y0, aggs):
    grid = (NPAD // ROW_BLK,)
    return pl.pallas_call(
        _sum_body,
        grid=grid,
        in_specs=[
            pl.BlockSpec((ROW_BLK, D_C), lambda i: (i, 0)),
            pl.BlockSpec((N_REL_C, ROW_BLK, D_C), lambda i: (0, i, 0)),
        ],
        out_specs=pl.BlockSpec((ROW_BLK, D_C), lambda i: (i, 0)),
        out_shape=jax.ShapeDtypeStruct((NPAD, D_C), jnp.float32),
    )(y0, aggs)


def _make_sc_aggregate():
    mesh = plsc.VectorSubcoreMesh(core_axis_name="c", subcore_axis_name="s",
                                  num_cores=2)

    @functools.partial(
        pl.kernel,
        out_type=jax.ShapeDtypeStruct((N_REL_C, NPAD, D_C), jnp.float32),
        mesh=mesh,
        compiler_params=pltpu.CompilerParams(needs_layout_passes=False),
        scratch_types=[
            pltpu.VMEM((CHUNK,), jnp.int32),
            pltpu.VMEM((CHUNK,), jnp.int32),
            pltpu.VMEM((CHUNK,), jnp.int32),
            pltpu.VMEM((CHUNK,), jnp.int32),
            pltpu.VMEM((CHUNK + G,), jnp.int32),
            pltpu.VMEM((CHUNK + G,), jnp.int32),
            pltpu.VMEM((2 * K * G, D_C), jnp.float32),
            pltpu.VMEM((RPW + 1, D_C), jnp.float32),
            pltpu.SemaphoreType.DMA,
            pltpu.SemaphoreType.DMA,
            pltpu.SemaphoreType.DMA,
            pltpu.SemaphoreType.DMA,
        ],
    )
    def sc_aggregate(h_hbm, ei_hbm, agg_hbm,
                     srcA, dstA, srcB, dstB, slist, llist,
                     ring, acc, semA, semB, semG0, semG1):
        cid = lax.axis_index("c")
        sid = lax.axis_index("s")
        wid = sid * 2 + cid
        lo = wid * RPW
        hi = lo + RPW

        ninf16 = jnp.full((16,), NEG_INF, jnp.float32)
        zero16 = jnp.zeros((16,), jnp.int32)
        trash16 = jnp.full((16,), RPW, jnp.int32)

        def issue_chunk(c, sbuf, dbuf, sem, soff, doff):
            pltpu.async_copy(ei_hbm.at[pl.ds(soff + c * CHUNK, CHUNK)],
                             sbuf, sem)
            pltpu.async_copy(ei_hbm.at[pl.ds(doff + c * CHUNK, CHUNK)],
                             dbuf, sem)

        def wait_chunk(sbuf, dbuf, sem):
            pltpu.make_async_copy(ei_hbm.at[pl.ds(0, CHUNK)], sbuf,
                                  sem).wait()
            pltpu.make_async_copy(ei_hbm.at[pl.ds(0, CHUNK)], dbuf,
                                  sem).wait()

        def rel_body(r, _):
            soff = (2 * r) * N_EDGES_C
            doff = soff + N_EDGES_C
            goff = r * NPAD

            def init_body(i, _):
                for k in range(NKS):
                    acc[i, pl.ds(k * 16, 16)] = ninf16
                return 0
            lax.fori_loop(0, RPW, init_body, 0)

            issue_chunk(0, srcA, dstA, semA, soff, doff)

            def process(c, sbuf, dbuf, sem, nsbuf, ndbuf, nsem):
                wait_chunk(sbuf, dbuf, sem)

                @pl.when(c + 1 < NCHUNK)
                def _():
                    issue_chunk(c + 1, nsbuf, ndbuf, nsem, soff, doff)

                def fbody(i, cnt):
                    dv0 = dbuf[pl.ds(i * 32, 16)]
                    dv1 = dbuf[pl.ds(i * 32 + 16, 16)]
                    m0 = (dv0 >= lo) & (dv0 < hi)
                    m1 = (dv1 >= lo) & (dv1 < hi)
                    sv0 = sbuf[pl.ds(i * 32, 16)] + goff
                    sv1 = sbuf[pl.ds(i * 32 + 16, 16)] + goff
                    plsc.store_compressed(llist.at[pl.ds(cnt, 16)],
                                          dv0 - lo, mask=m0)
                    plsc.store_compressed(slist.at[pl.ds(cnt, 16)], sv0,
                                          mask=m0)
                    pc0 = plsc.all_reduce_population_count(m0)
                    cnt1 = cnt + pc0[0]
                    plsc.store_compressed(llist.at[pl.ds(cnt1, 16)],
                                          dv1 - lo, mask=m1)
                    plsc.store_compressed(slist.at[pl.ds(cnt1, 16)], sv1,
                                          mask=m1)
                    pc1 = plsc.all_reduce_population_count(m1)
                    return cnt1 + pc1[0]

                n = lax.fori_loop(0, CHUNK // 32, fbody, jnp.int32(0))

                for t in range(G // 16):
                    slist[pl.ds(n + t * 16, 16)] = zero16
                    llist[pl.ds(n + t * 16, 16)] = trash16
                nblk = (n + (G - 1)) // G
                nsb = (nblk + (K - 1)) // K

                def fire_sb(s):
                    m = jnp.minimum(K, nblk - s * K)
                    sem_sel = lax.rem(s, 2)
                    base = sem_sel * (K * G)

                    def fire(b, _):
                        blk = s * K + b
                        dst = ring.at[pl.ds(base + b * G, G)]

                        @pl.when(sem_sel == 0)
                        def _():
                            pltpu.async_copy(
                                h_hbm.at[slist.at[pl.ds(blk * G, G)]],
                                dst, semG0)

                        @pl.when(sem_sel == 1)
                        def _():
                            pltpu.async_copy(
                                h_hbm.at[slist.at[pl.ds(blk * G, G)]],
                                dst, semG1)
                        return 0

                    lax.fori_loop(0, m, fire, 0)

                def sb_body(s, _):
                    @pl.when(s + 1 < nsb)
                    def _():
                        fire_sb(s + 1)

                    m = jnp.minimum(K, nblk - s * K)
                    sem_sel = lax.rem(s, 2)
                    base = sem_sel * (K * G)

                    def drain(b, _):
                        @pl.when(sem_sel == 0)
                        def _():
                            pltpu.make_async_copy(
                                h_hbm.at[slist.at[pl.ds(0, G)]],
                                ring.at[pl.ds(0, G)], semG0).wait()

                        @pl.when(sem_sel == 1)
                        def _():
                            pltpu.make_async_copy(
                                h_hbm.at[slist.at[pl.ds(0, G)]],
                                ring.at[pl.ds(0, G)], semG1).wait()
                        return 0

                    lax.fori_loop(0, m, drain, 0)

                    def mblk(b, _):
                        blk = s * K + b
                        roff = base + b * G
                        for g in range(G // 16):
                            lvec = llist[pl.ds(blk * G + g * 16, 16)]
                            for e in range(16):
                                dloc = lvec[e]
                                ri = roff + g * 16 + e
                                rv = [ring[ri, pl.ds(k * 16, 16)]
                                      for k in range(NKS)]
                                av = [acc[dloc, pl.ds(k * 16, 16)]
                                      for k in range(NKS)]
                                for k in range(NKS):
                                    acc[dloc, pl.ds(k * 16, 16)] = (
                                        jnp.maximum(av[k], rv[k]))
                        return 0

                    lax.fori_loop(0, m, mblk, 0)
                    return 0

                @pl.when(nsb > 0)
                def _():
                    fire_sb(0)
                lax.fori_loop(0, nsb, sb_body, 0)

            def pair_body(cc, _):
                c0 = 2 * cc
                process(c0, srcA, dstA, semA, srcB, dstB, semB)
                process(c0 + 1, srcB, dstB, semB, srcA, dstA, semA)
                return 0

            lax.fori_loop(0, NCHUNK // 2, pair_body, 0)

            def finish(i, _):
                for k in range(NKS):
                    sl = pl.ds(k * 16, 16)
                    a = acc[i, sl]
                    acc[i, sl] = jnp.where(a == NEG_INF, jnp.float32(0.0), a)
                return 0
            lax.fori_loop(0, RPW, finish, 0)

            pltpu.sync_copy(acc.at[pl.ds(0, RPW)],
                            agg_hbm.at[r, pl.ds(lo, RPW)])
            return 0

        lax.fori_loop(0, N_REL_C, rel_body, 0)

    return sc_aggregate


_sc_aggregate = None


def kernel(x, edge_indices_list, W_root, b_root, W_rel):
    global _sc_aggregate
    if _sc_aggregate is None:
        _sc_aggregate = _make_sc_aggregate()
    x_pad = jnp.pad(x, ((0, NPAD - N_NODES_C), (0, 0)))
    w_all = jnp.concatenate([W_root[None], W_rel], axis=0)
    b_all = jnp.concatenate(
        [b_root[None], jnp.zeros((3, D_C), jnp.float32)], 0
    ).reshape(4, 1, D_C)
    y = _linear_all(x_pad, w_all, b_all)
    h = y[1:].reshape(3 * NPAD, D_C)
    ei = edge_indices_list.astype(jnp.int32).reshape(-1)
    aggs = _sc_aggregate(h, ei)
    out_pad = _final_sum(y[0], aggs)
    return out_pad[:N_NODES_C]


# speedup vs baseline: 2.4218x; 1.3983x over previous
import functools

import jax
import jax.numpy as jnp
from jax import lax
from jax.experimental import pallas as pl
from jax.experimental.pallas import tpu as pltpu
from jax.experimental.pallas import tpu_sc as plsc

N_NODES_C = 10000
N_REL_C = 3
N_EDGES_C = 320000
D_C = 128
NPAD = 10240
ROW_BLK = 1024
NW = 32
RPW = NPAD // NW
CHUNK = 6400
NCHUNK = N_EDGES_C // CHUNK
G = 16
K = 4
NEG_INF = float("-inf")
NKS = D_C // 32


def _matmul_body(x_ref, w_ref, b_ref, y_ref):
    xb = x_ref[...]
    w = w_ref[0]
    acc = lax.dot_general(xb, w, (((1,), (1,)), ((), ())),
                          preferred_element_type=jnp.float32)
    y_ref[0] = acc + b_ref[0]


def _linear_all(x_pad, w_all, b_all):
    grid = (4, NPAD // ROW_BLK)
    return pl.pallas_call(
        _matmul_body,
        grid=grid,
        in_specs=[
            pl.BlockSpec((ROW_BLK, D_C), lambda j, i: (i, 0)),
            pl.BlockSpec((1, D_C, D_C), lambda j, i: (j, 0, 0)),
            pl.BlockSpec((1, 1, D_C), lambda j, i: (j, 0, 0)),
        ],
        out_specs=pl.BlockSpec((1, ROW_BLK, D_C), lambda j, i: (j, i, 0)),
        out_shape=jax.ShapeDtypeStruct((4, NPAD, D_C), jnp.float32),
    )(x_pad, w_all, b_all)


def _sum_body(y0_ref, a_ref, o_ref):
    a = a_ref[...].astype(jnp.float32)
    o_ref[...] = y0_ref[...] + a[0] + a[1] + a[2]


def _final_sum(y0, aggs):
    grid = (NPAD // ROW_BLK,)
    return pl.pallas_call(
        _sum_body,
        grid=grid,
        in_specs=[
            pl.BlockSpec((ROW_BLK, D_C), lambda i: (i, 0)),
            pl.BlockSpec((N_REL_C, ROW_BLK, D_C), lambda i: (0, i, 0)),
        ],
        out_specs=pl.BlockSpec((ROW_BLK, D_C), lambda i: (i, 0)),
        out_shape=jax.ShapeDtypeStruct((NPAD, D_C), jnp.float32),
    )(y0, aggs)


def _make_sc_aggregate():
    mesh = plsc.VectorSubcoreMesh(core_axis_name="c", subcore_axis_name="s",
                                  num_cores=2)

    @functools.partial(
        pl.kernel,
        out_type=jax.ShapeDtypeStruct((N_REL_C, NPAD, D_C), jnp.bfloat16),
        mesh=mesh,
        compiler_params=pltpu.CompilerParams(needs_layout_passes=False,
                                             use_tc_tiling_on_sc=False),
        scratch_types=[
            pltpu.VMEM((CHUNK,), jnp.int32),
            pltpu.VMEM((CHUNK,), jnp.int32),
            pltpu.VMEM((CHUNK,), jnp.int32),
            pltpu.VMEM((CHUNK,), jnp.int32),
            pltpu.VMEM((CHUNK + G,), jnp.int32),
            pltpu.VMEM((CHUNK + G,), jnp.int32),
            pltpu.VMEM((2 * K * G, D_C // 2), jnp.int32),
            pltpu.VMEM((RPW + 1, D_C), jnp.bfloat16),
            pltpu.SemaphoreType.DMA,
            pltpu.SemaphoreType.DMA,
            pltpu.SemaphoreType.DMA,
            pltpu.SemaphoreType.DMA,
        ],
    )
    def sc_aggregate(h_hbm, ei_hbm, agg_hbm,
                     srcA, dstA, srcB, dstB, slist, llist,
                     ring, acc, semA, semB, semG0, semG1):
        cid = lax.axis_index("c")
        sid = lax.axis_index("s")
        wid = sid * 2 + cid
        lo = wid * RPW
        hi = lo + RPW

        ninf32 = jnp.full((32,), NEG_INF, jnp.bfloat16)
        zero16 = jnp.zeros((16,), jnp.int32)
        trash16 = jnp.full((16,), RPW, jnp.int32)

        def issue_chunk(c, sbuf, dbuf, sem, soff, doff):
            pltpu.async_copy(ei_hbm.at[pl.ds(soff + c * CHUNK, CHUNK)],
                             sbuf, sem)
            pltpu.async_copy(ei_hbm.at[pl.ds(doff + c * CHUNK, CHUNK)],
                             dbuf, sem)

        def wait_chunk(sbuf, dbuf, sem):
            pltpu.make_async_copy(ei_hbm.at[pl.ds(0, CHUNK)], sbuf,
                                  sem).wait()
            pltpu.make_async_copy(ei_hbm.at[pl.ds(0, CHUNK)], dbuf,
                                  sem).wait()

        def rel_body(r, _):
            soff = (2 * r) * N_EDGES_C
            doff = soff + N_EDGES_C
            goff = r * NPAD

            def init_body(i, _):
                for k in range(NKS):
                    acc[i, pl.ds(k * 32, 32)] = ninf32
                return 0
            lax.fori_loop(0, RPW, init_body, 0)

            issue_chunk(0, srcA, dstA, semA, soff, doff)

            def process(c, sbuf, dbuf, sem, nsbuf, ndbuf, nsem):
                wait_chunk(sbuf, dbuf, sem)

                @pl.when(c + 1 < NCHUNK)
                def _():
                    issue_chunk(c + 1, nsbuf, ndbuf, nsem, soff, doff)

                def fbody(i, cnt):
                    dv0 = dbuf[pl.ds(i * 32, 16)]
                    dv1 = dbuf[pl.ds(i * 32 + 16, 16)]
                    m0 = (dv0 >= lo) & (dv0 < hi)
                    m1 = (dv1 >= lo) & (dv1 < hi)
                    sv0 = sbuf[pl.ds(i * 32, 16)] + goff
                    sv1 = sbuf[pl.ds(i * 32 + 16, 16)] + goff
                    plsc.store_compressed(llist.at[pl.ds(cnt, 16)],
                                          dv0 - lo, mask=m0)
                    plsc.store_compressed(slist.at[pl.ds(cnt, 16)], sv0,
                                          mask=m0)
                    pc0 = plsc.all_reduce_population_count(m0)
                    cnt1 = cnt + pc0[0]
                    plsc.store_compressed(llist.at[pl.ds(cnt1, 16)],
                                          dv1 - lo, mask=m1)
                    plsc.store_compressed(slist.at[pl.ds(cnt1, 16)], sv1,
                                          mask=m1)
                    pc1 = plsc.all_reduce_population_count(m1)
                    return cnt1 + pc1[0]

                n = lax.fori_loop(0, CHUNK // 32, fbody, jnp.int32(0))

                for t in range(G // 16):
                    slist[pl.ds(n + t * 16, 16)] = zero16
                    llist[pl.ds(n + t * 16, 16)] = trash16
                nblk = (n + (G - 1)) // G
                nsb = (nblk + (K - 1)) // K

                def fire_sb(s):
                    m = jnp.minimum(K, nblk - s * K)
                    sem_sel = lax.rem(s, 2)
                    base = sem_sel * (K * G)

                    def fire(b, _):
                        blk = s * K + b
                        dst = ring.at[pl.ds(base + b * G, G)]

                        @pl.when(sem_sel == 0)
                        def _():
                            pltpu.async_copy(
                                h_hbm.at[slist.at[pl.ds(blk * G, G)]],
                                dst, semG0)

                        @pl.when(sem_sel == 1)
                        def _():
                            pltpu.async_copy(
                                h_hbm.at[slist.at[pl.ds(blk * G, G)]],
                                dst, semG1)
                        return 0

                    lax.fori_loop(0, m, fire, 0)

                def sb_body(s, _):
                    @pl.when(s + 1 < nsb)
                    def _():
                        fire_sb(s + 1)

                    m = jnp.minimum(K, nblk - s * K)
                    sem_sel = lax.rem(s, 2)
                    base = sem_sel * (K * G)

                    def drain(b, _):
                        @pl.when(sem_sel == 0)
                        def _():
                            pltpu.make_async_copy(
                                h_hbm.at[slist.at[pl.ds(0, G)]],
                                ring.at[pl.ds(0, G)], semG0).wait()

                        @pl.when(sem_sel == 1)
                        def _():
                            pltpu.make_async_copy(
                                h_hbm.at[slist.at[pl.ds(0, G)]],
                                ring.at[pl.ds(0, G)], semG1).wait()
                        return 0

                    lax.fori_loop(0, m, drain, 0)

                    def mblk(b, _):
                        blk = s * K + b
                        roff = base + b * G
                        for g in range(G // 16):
                            lvec = llist[pl.ds(blk * G + g * 16, 16)]
                            for e in range(16):
                                dloc = lvec[e]
                                ri = roff + g * 16 + e
                                rv = [plsc.bitcast(
                                          ring[ri, pl.ds(k * 16, 16)],
                                          jnp.bfloat16)
                                      for k in range(NKS)]
                                av = [acc[dloc, pl.ds(k * 32, 32)]
                                      for k in range(NKS)]
                                for k in range(NKS):
                                    acc[dloc, pl.ds(k * 32, 32)] = (
                                        jnp.maximum(av[k], rv[k]))
                        return 0

                    lax.fori_loop(0, m, mblk, 0)
                    return 0

                @pl.when(nsb > 0)
                def _():
                    fire_sb(0)
                lax.fori_loop(0, nsb, sb_body, 0)

            def pair_body(cc, _):
                c0 = 2 * cc
                process(c0, srcA, dstA, semA, srcB, dstB, semB)
                process(c0 + 1, srcB, dstB, semB, srcA, dstA, semA)
                return 0

            lax.fori_loop(0, NCHUNK // 2, pair_body, 0)

            def finish(i, _):
                for k in range(NKS):
                    sl = pl.ds(k * 32, 32)
                    a = acc[i, sl]
                    acc[i, sl] = jnp.where(a == ninf32, jnp.bfloat16(0.0), a)
                return 0
            lax.fori_loop(0, RPW, finish, 0)

            pltpu.sync_copy(acc.at[pl.ds(0, RPW)],
                            agg_hbm.at[r, pl.ds(lo, RPW)])
            return 0

        lax.fori_loop(0, N_REL_C, rel_body, 0)

    return sc_aggregate


_sc_aggregate = None


def kernel(x, edge_indices_list, W_root, b_root, W_rel):
    global _sc_aggregate
    if _sc_aggregate is None:
        _sc_aggregate = _make_sc_aggregate()
    x_pad = jnp.pad(x, ((0, NPAD - N_NODES_C), (0, 0)))
    w_all = jnp.concatenate([W_root[None], W_rel], axis=0)
    b_all = jnp.concatenate(
        [b_root[None], jnp.zeros((3, D_C), jnp.float32)], 0
    ).reshape(4, 1, D_C)
    y = _linear_all(x_pad, w_all, b_all)
    h_bf = y[1:].reshape(3 * NPAD, D_C).astype(jnp.bfloat16)
    h = lax.bitcast_convert_type(
        h_bf.reshape(3 * NPAD, D_C // 2, 2), jnp.int32)
    ei = edge_indices_list.astype(jnp.int32).reshape(-1)
    aggs = _sc_aggregate(h, ei)
    out_pad = _final_sum(y[0], aggs)
    return out_pad[:N_NODES_C]


# speedup vs baseline: 2.4970x; 1.0311x over previous
import functools

import jax
import jax.numpy as jnp
from jax import lax
from jax.experimental import pallas as pl
from jax.experimental.pallas import tpu as pltpu
from jax.experimental.pallas import tpu_sc as plsc

N_NODES_C = 10000
N_REL_C = 3
N_EDGES_C = 320000
D_C = 128
NPAD = 10240
ROW_BLK = 1024
NW = 32
RPW = NPAD // NW
CHUNK = 6400
NCHUNK = N_EDGES_C // CHUNK
G = 16
K = 8
NEG_INF = float("-inf")
NKS = D_C // 32


def _matmul_body(x_ref, w_ref, b_ref, y_ref):
    xb = x_ref[...]
    w = w_ref[0]
    acc = lax.dot_general(xb, w, (((1,), (1,)), ((), ())),
                          preferred_element_type=jnp.float32)
    y_ref[0] = acc + b_ref[0]


def _linear_all(x_pad, w_all, b_all):
    grid = (4, NPAD // ROW_BLK)
    return pl.pallas_call(
        _matmul_body,
        grid=grid,
        in_specs=[
            pl.BlockSpec((ROW_BLK, D_C), lambda j, i: (i, 0)),
            pl.BlockSpec((1, D_C, D_C), lambda j, i: (j, 0, 0)),
            pl.BlockSpec((1, 1, D_C), lambda j, i: (j, 0, 0)),
        ],
        out_specs=pl.BlockSpec((1, ROW_BLK, D_C), lambda j, i: (j, i, 0)),
        out_shape=jax.ShapeDtypeStruct((4, NPAD, D_C), jnp.float32),
    )(x_pad, w_all, b_all)


def _sum_body(y0_ref, a_ref, o_ref):
    a = a_ref[...].astype(jnp.float32)
    o_ref[...] = y0_ref[...] + a[0] + a[1] + a[2]


def _final_sum(y0, aggs):
    grid = (NPAD // ROW_BLK,)
    return pl.pallas_call(
        _sum_body,
        grid=grid,
        in_specs=[
            pl.BlockSpec((ROW_BLK, D_C), lambda i: (i, 0)),
            pl.BlockSpec((N_REL_C, ROW_BLK, D_C), lambda i: (0, i, 0)),
        ],
        out_specs=pl.BlockSpec((ROW_BLK, D_C), lambda i: (i, 0)),
        out_shape=jax.ShapeDtypeStruct((NPAD, D_C), jnp.float32),
    )(y0, aggs)


def _make_sc_aggregate():
    mesh = plsc.VectorSubcoreMesh(core_axis_name="c", subcore_axis_name="s",
                                  num_cores=2)

    @functools.partial(
        pl.kernel,
        out_type=jax.ShapeDtypeStruct((N_REL_C, NPAD, D_C), jnp.bfloat16),
        mesh=mesh,
        compiler_params=pltpu.CompilerParams(needs_layout_passes=False,
                                             use_tc_tiling_on_sc=False),
        scratch_types=[
            pltpu.VMEM((CHUNK,), jnp.int32),
            pltpu.VMEM((CHUNK,), jnp.int32),
            pltpu.VMEM((CHUNK,), jnp.int32),
            pltpu.VMEM((CHUNK,), jnp.int32),
            pltpu.VMEM((CHUNK + G,), jnp.int32),
            pltpu.VMEM((CHUNK + G,), jnp.int32),
            pltpu.VMEM((2 * K * G, D_C // 2), jnp.int32),
            pltpu.VMEM((RPW + 1, D_C), jnp.bfloat16),
            pltpu.SemaphoreType.DMA,
            pltpu.SemaphoreType.DMA,
            pltpu.SemaphoreType.DMA,
            pltpu.SemaphoreType.DMA,
        ],
    )
    def sc_aggregate(h_hbm, ei_hbm, agg_hbm,
                     srcA, dstA, srcB, dstB, slist, llist,
                     ring, acc, semA, semB, semG0, semG1):
        cid = lax.axis_index("c")
        sid = lax.axis_index("s")
        wid = sid * 2 + cid
        lo = wid * RPW
        hi = lo + RPW

        ninf32 = jnp.full((32,), NEG_INF, jnp.bfloat16)
        zero16 = jnp.zeros((16,), jnp.int32)
        trash16 = jnp.full((16,), RPW, jnp.int32)

        def issue_chunk(c, sbuf, dbuf, sem, soff, doff):
            pltpu.async_copy(ei_hbm.at[pl.ds(soff + c * CHUNK, CHUNK)],
                             sbuf, sem)
            pltpu.async_copy(ei_hbm.at[pl.ds(doff + c * CHUNK, CHUNK)],
                             dbuf, sem)

        def wait_chunk(sbuf, dbuf, sem):
            pltpu.make_async_copy(ei_hbm.at[pl.ds(0, CHUNK)], sbuf,
                                  sem).wait()
            pltpu.make_async_copy(ei_hbm.at[pl.ds(0, CHUNK)], dbuf,
                                  sem).wait()

        def rel_body(r, _):
            soff = (2 * r) * N_EDGES_C
            doff = soff + N_EDGES_C
            goff = r * NPAD

            def init_body(i, _):
                for k in range(NKS):
                    acc[i, pl.ds(k * 32, 32)] = ninf32
                return 0
            lax.fori_loop(0, RPW, init_body, 0)

            issue_chunk(0, srcA, dstA, semA, soff, doff)

            def process(c, sbuf, dbuf, sem, nsbuf, ndbuf, nsem):
                wait_chunk(sbuf, dbuf, sem)

                @pl.when(c + 1 < NCHUNK)
                def _():
                    issue_chunk(c + 1, nsbuf, ndbuf, nsem, soff, doff)

                def fbody(i, cnt):
                    dv0 = dbuf[pl.ds(i * 32, 16)]
                    dv1 = dbuf[pl.ds(i * 32 + 16, 16)]
                    m0 = (dv0 >= lo) & (dv0 < hi)
                    m1 = (dv1 >= lo) & (dv1 < hi)
                    sv0 = sbuf[pl.ds(i * 32, 16)] + goff
                    sv1 = sbuf[pl.ds(i * 32 + 16, 16)] + goff
                    plsc.store_compressed(llist.at[pl.ds(cnt, 16)],
                                          dv0 - lo, mask=m0)
                    plsc.store_compressed(slist.at[pl.ds(cnt, 16)], sv0,
                                          mask=m0)
                    pc0 = plsc.all_reduce_population_count(m0)
                    cnt1 = cnt + pc0[0]
                    plsc.store_compressed(llist.at[pl.ds(cnt1, 16)],
                                          dv1 - lo, mask=m1)
                    plsc.store_compressed(slist.at[pl.ds(cnt1, 16)], sv1,
                                          mask=m1)
                    pc1 = plsc.all_reduce_population_count(m1)
                    return cnt1 + pc1[0]

                n = lax.fori_loop(0, CHUNK // 32, fbody, jnp.int32(0))

                for t in range(G // 16):
                    slist[pl.ds(n + t * 16, 16)] = zero16
                    llist[pl.ds(n + t * 16, 16)] = trash16
                nblk = (n + (G - 1)) // G
                nsb = (nblk + (K - 1)) // K

                def fire_sb(s):
                    m = jnp.minimum(K, nblk - s * K)
                    sem_sel = lax.rem(s, 2)
                    base = sem_sel * (K * G)

                    def fire(b, _):
                        blk = s * K + b
                        dst = ring.at[pl.ds(base + b * G, G)]

                        @pl.when(sem_sel == 0)
                        def _():
                            pltpu.async_copy(
                                h_hbm.at[slist.at[pl.ds(blk * G, G)]],
                                dst, semG0)

                        @pl.when(sem_sel == 1)
                        def _():
                            pltpu.async_copy(
                                h_hbm.at[slist.at[pl.ds(blk * G, G)]],
                                dst, semG1)
                        return 0

                    lax.fori_loop(0, m, fire, 0)

                def sb_body(s, _):
                    @pl.when(s + 1 < nsb)
                    def _():
                        fire_sb(s + 1)

                    m = jnp.minimum(K, nblk - s * K)
                    sem_sel = lax.rem(s, 2)
                    base = sem_sel * (K * G)

                    def drain(b, _):
                        @pl.when(sem_sel == 0)
                        def _():
                            pltpu.make_async_copy(
                                h_hbm.at[slist.at[pl.ds(0, G)]],
                                ring.at[pl.ds(0, G)], semG0).wait()

                        @pl.when(sem_sel == 1)
                        def _():
                            pltpu.make_async_copy(
                                h_hbm.at[slist.at[pl.ds(0, G)]],
                                ring.at[pl.ds(0, G)], semG1).wait()
                        return 0

                    lax.fori_loop(0, m, drain, 0)

                    def mblk(b, _):
                        blk = s * K + b
                        roff = base + b * G
                        for g in range(G // 16):
                            lvec = llist[pl.ds(blk * G + g * 16, 16)]
                            for e in range(16):
                                dloc = lvec[e]
                                ri = roff + g * 16 + e
                                rv = [plsc.bitcast(
                                          ring[ri, pl.ds(k * 16, 16)],
                                          jnp.bfloat16)
                                      for k in range(NKS)]
                                av = [acc[dloc, pl.ds(k * 32, 32)]
                                      for k in range(NKS)]
                                for k in range(NKS):
                                    acc[dloc, pl.ds(k * 32, 32)] = (
                                        jnp.maximum(av[k], rv[k]))
                        return 0

                    lax.fori_loop(0, m, mblk, 0)
                    return 0

                @pl.when(nsb > 0)
                def _():
                    fire_sb(0)
                lax.fori_loop(0, nsb, sb_body, 0)

            def pair_body(cc, _):
                c0 = 2 * cc
                process(c0, srcA, dstA, semA, srcB, dstB, semB)
                process(c0 + 1, srcB, dstB, semB, srcA, dstA, semA)
                return 0

            lax.fori_loop(0, NCHUNK // 2, pair_body, 0)

            def finish(i, _):
                for k in range(NKS):
                    sl = pl.ds(k * 32, 32)
                    a = acc[i, sl]
                    acc[i, sl] = jnp.where(a == ninf32, jnp.bfloat16(0.0), a)
                return 0
            lax.fori_loop(0, RPW, finish, 0)

            pltpu.sync_copy(acc.at[pl.ds(0, RPW)],
                            agg_hbm.at[r, pl.ds(lo, RPW)])
            return 0

        lax.fori_loop(0, N_REL_C, rel_body, 0)

    return sc_aggregate


_sc_aggregate = None


def kernel(x, edge_indices_list, W_root, b_root, W_rel):
    global _sc_aggregate
    if _sc_aggregate is None:
        _sc_aggregate = _make_sc_aggregate()
    x_pad = jnp.pad(x, ((0, NPAD - N_NODES_C), (0, 0)))
    w_all = jnp.concatenate([W_root[None], W_rel], axis=0)
    b_all = jnp.concatenate(
        [b_root[None], jnp.zeros((3, D_C), jnp.float32)], 0
    ).reshape(4, 1, D_C)
    y = _linear_all(x_pad, w_all, b_all)
    h_bf = y[1:].reshape(3 * NPAD, D_C).astype(jnp.bfloat16)
    h = lax.bitcast_convert_type(
        h_bf.reshape(3 * NPAD, D_C // 2, 2), jnp.int32)
    ei = edge_indices_list.astype(jnp.int32).reshape(-1)
    aggs = _sc_aggregate(h, ei)
    out_pad = _final_sum(y[0], aggs)
    return out_pad[:N_NODES_C]


# speedup vs baseline: 2.7903x; 1.1174x over previous
import functools

import jax
import jax.numpy as jnp
from jax import lax
from jax.experimental import pallas as pl
from jax.experimental.pallas import tpu as pltpu
from jax.experimental.pallas import tpu_sc as plsc

N_NODES_C = 10000
N_REL_C = 3
N_EDGES_C = 320000
D_C = 128
NPAD = 10240
ROW_BLK = 1024
NW = 32
RPW = NPAD // NW
CHUNK = 6400
NCHUNK = N_EDGES_C // CHUNK
G = 16
K = 8
NEG_INF = float("-inf")
NKS = D_C // 32


def _matmul_body(x_ref, w_ref, b_ref, y_ref):
    xb = x_ref[...]
    w = w_ref[0]
    acc = lax.dot_general(xb, w, (((1,), (1,)), ((), ())),
                          preferred_element_type=jnp.float32)
    y_ref[0] = acc + b_ref[0]


def _linear_all(x_pad, w_all, b_all):
    grid = (4, NPAD // ROW_BLK)
    return pl.pallas_call(
        _matmul_body,
        grid=grid,
        in_specs=[
            pl.BlockSpec((ROW_BLK, D_C), lambda j, i: (i, 0)),
            pl.BlockSpec((1, D_C, D_C), lambda j, i: (j, 0, 0)),
            pl.BlockSpec((1, 1, D_C), lambda j, i: (j, 0, 0)),
        ],
        out_specs=pl.BlockSpec((1, ROW_BLK, D_C), lambda j, i: (j, i, 0)),
        out_shape=jax.ShapeDtypeStruct((4, NPAD, D_C), jnp.float32),
    )(x_pad, w_all, b_all)


def _sum_body(y0_ref, a_ref, o_ref):
    a = a_ref[...].astype(jnp.float32)
    o_ref[...] = y0_ref[...] + a[0] + a[1] + a[2]


def _final_sum(y0, aggs):
    grid = (NPAD // ROW_BLK,)
    return pl.pallas_call(
        _sum_body,
        grid=grid,
        in_specs=[
            pl.BlockSpec((ROW_BLK, D_C), lambda i: (i, 0)),
            pl.BlockSpec((N_REL_C, ROW_BLK, D_C), lambda i: (0, i, 0)),
        ],
        out_specs=pl.BlockSpec((ROW_BLK, D_C), lambda i: (i, 0)),
        out_shape=jax.ShapeDtypeStruct((NPAD, D_C), jnp.float32),
    )(y0, aggs)


def _make_sc_aggregate():
    mesh = plsc.VectorSubcoreMesh(core_axis_name="c", subcore_axis_name="s",
                                  num_cores=2)

    @functools.partial(
        pl.kernel,
        out_type=jax.ShapeDtypeStruct((N_REL_C, NPAD, D_C), jnp.bfloat16),
        mesh=mesh,
        compiler_params=pltpu.CompilerParams(needs_layout_passes=False,
                                             use_tc_tiling_on_sc=False),
        scratch_types=[
            pltpu.VMEM((CHUNK,), jnp.int32),
            pltpu.VMEM((CHUNK,), jnp.int32),
            pltpu.VMEM((CHUNK,), jnp.int32),
            pltpu.VMEM((CHUNK,), jnp.int32),
            pltpu.VMEM((CHUNK + G,), jnp.int32),
            pltpu.VMEM((CHUNK + G,), jnp.int32),
            pltpu.VMEM((2 * K * G, D_C // 2), jnp.int32),
            pltpu.VMEM((RPW + 1, D_C), jnp.bfloat16),
            pltpu.SemaphoreType.DMA,
            pltpu.SemaphoreType.DMA,
            pltpu.SemaphoreType.DMA,
            pltpu.SemaphoreType.DMA,
        ],
    )
    def sc_aggregate(h_hbm, ei_hbm, agg_hbm,
                     srcA, dstA, srcB, dstB, slist, llist,
                     ring, acc, semA, semB, semG0, semG1):
        cid = lax.axis_index("c")
        sid = lax.axis_index("s")
        wid = sid * 2 + cid
        lo = wid * RPW
        hi = lo + RPW

        ninf32 = jnp.full((32,), NEG_INF, jnp.bfloat16)
        zero16 = jnp.zeros((16,), jnp.int32)
        trash16 = jnp.full((16,), RPW, jnp.int32)

        def issue_chunk(c, sbuf, dbuf, sem, soff, doff):
            pltpu.async_copy(ei_hbm.at[pl.ds(soff + c * CHUNK, CHUNK)],
                             sbuf, sem)
            pltpu.async_copy(ei_hbm.at[pl.ds(doff + c * CHUNK, CHUNK)],
                             dbuf, sem)

        def wait_chunk(sbuf, dbuf, sem):
            pltpu.make_async_copy(ei_hbm.at[pl.ds(0, CHUNK)], sbuf,
                                  sem).wait()
            pltpu.make_async_copy(ei_hbm.at[pl.ds(0, CHUNK)], dbuf,
                                  sem).wait()

        def rel_body(r, _):
            soff = (2 * r) * N_EDGES_C
            doff = soff + N_EDGES_C
            goff = r * NPAD

            def init_body(i, _):
                for k in range(NKS):
                    acc[i, pl.ds(k * 32, 32)] = ninf32
                return 0
            lax.fori_loop(0, RPW, init_body, 0)

            issue_chunk(0, srcA, dstA, semA, soff, doff)

            def process(c, sbuf, dbuf, sem, nsbuf, ndbuf, nsem):
                wait_chunk(sbuf, dbuf, sem)

                @pl.when(c + 1 < NCHUNK)
                def _():
                    issue_chunk(c + 1, nsbuf, ndbuf, nsem, soff, doff)

                def fbody(i, cnt):
                    dv = [dbuf[pl.ds(i * 64 + 16 * u, 16)] for u in range(4)]
                    sv = [sbuf[pl.ds(i * 64 + 16 * u, 16)] + goff
                          for u in range(4)]
                    ms = [(d >= lo) & (d < hi) for d in dv]
                    pc = [plsc.all_reduce_population_count(m) for m in ms]
                    for u in range(4):
                        plsc.store_compressed(llist.at[pl.ds(cnt, 16)],
                                              dv[u] - lo, mask=ms[u])
                        plsc.store_compressed(slist.at[pl.ds(cnt, 16)],
                                              sv[u], mask=ms[u])
                        cnt = cnt + pc[u][0]
                    return cnt

                n = lax.fori_loop(0, CHUNK // 64, fbody, jnp.int32(0))

                for t in range(G // 16):
                    slist[pl.ds(n + t * 16, 16)] = zero16
                    llist[pl.ds(n + t * 16, 16)] = trash16
                nblk = (n + (G - 1)) // G
                nsb = (nblk + (K - 1)) // K

                def fire_sb(s):
                    m = jnp.minimum(K, nblk - s * K)
                    sem_sel = lax.rem(s, 2)
                    base = sem_sel * (K * G)

                    def fire(b, _):
                        blk = s * K + b
                        dst = ring.at[pl.ds(base + b * G, G)]

                        @pl.when(sem_sel == 0)
                        def _():
                            pltpu.async_copy(
                                h_hbm.at[slist.at[pl.ds(blk * G, G)]],
                                dst, semG0)

                        @pl.when(sem_sel == 1)
                        def _():
                            pltpu.async_copy(
                                h_hbm.at[slist.at[pl.ds(blk * G, G)]],
                                dst, semG1)
                        return 0

                    lax.fori_loop(0, m, fire, 0)

                def sb_body(s, _):
                    @pl.when(s + 1 < nsb)
                    def _():
                        fire_sb(s + 1)

                    m = jnp.minimum(K, nblk - s * K)
                    sem_sel = lax.rem(s, 2)
                    base = sem_sel * (K * G)

                    def drain(b, _):
                        @pl.when(sem_sel == 0)
                        def _():
                            pltpu.make_async_copy(
                                h_hbm.at[slist.at[pl.ds(0, G)]],
                                ring.at[pl.ds(0, G)], semG0).wait()

                        @pl.when(sem_sel == 1)
                        def _():
                            pltpu.make_async_copy(
                                h_hbm.at[slist.at[pl.ds(0, G)]],
                                ring.at[pl.ds(0, G)], semG1).wait()
                        return 0

                    lax.fori_loop(0, m, drain, 0)

                    def mblk(b, _):
                        blk = s * K + b
                        roff = base + b * G
                        for g in range(G // 16):
                            lvec = llist[pl.ds(blk * G + g * 16, 16)]
                            for e in range(16):
                                dloc = lvec[e]
                                ri = roff + g * 16 + e
                                rv = [plsc.bitcast(
                                          ring[ri, pl.ds(k * 16, 16)],
                                          jnp.bfloat16)
                                      for k in range(NKS)]
                                av = [acc[dloc, pl.ds(k * 32, 32)]
                                      for k in range(NKS)]
                                for k in range(NKS):
                                    acc[dloc, pl.ds(k * 32, 32)] = (
                                        jnp.maximum(av[k], rv[k]))
                        return 0

                    lax.fori_loop(0, m, mblk, 0)
                    return 0

                @pl.when(nsb > 0)
                def _():
                    fire_sb(0)
                lax.fori_loop(0, nsb, sb_body, 0)

            def pair_body(cc, _):
                c0 = 2 * cc
                process(c0, srcA, dstA, semA, srcB, dstB, semB)
                process(c0 + 1, srcB, dstB, semB, srcA, dstA, semA)
                return 0

            lax.fori_loop(0, NCHUNK // 2, pair_body, 0)

            def finish(i, _):
                for k in range(NKS):
                    sl = pl.ds(k * 32, 32)
                    a = acc[i, sl]
                    acc[i, sl] = jnp.where(a == ninf32, jnp.bfloat16(0.0), a)
                return 0
            lax.fori_loop(0, RPW, finish, 0)

            pltpu.sync_copy(acc.at[pl.ds(0, RPW)],
                            agg_hbm.at[r, pl.ds(lo, RPW)])
            return 0

        lax.fori_loop(0, N_REL_C, rel_body, 0)

    return sc_aggregate


_sc_aggregate = None


def kernel(x, edge_indices_list, W_root, b_root, W_rel):
    global _sc_aggregate
    if _sc_aggregate is None:
        _sc_aggregate = _make_sc_aggregate()
    x_pad = jnp.pad(x, ((0, NPAD - N_NODES_C), (0, 0)))
    w_all = jnp.concatenate([W_root[None], W_rel], axis=0)
    b_all = jnp.concatenate(
        [b_root[None], jnp.zeros((3, D_C), jnp.float32)], 0
    ).reshape(4, 1, D_C)
    y = _linear_all(x_pad, w_all, b_all)
    h_bf = y[1:].reshape(3 * NPAD, D_C).astype(jnp.bfloat16)
    h = lax.bitcast_convert_type(
        h_bf.reshape(3 * NPAD, D_C // 2, 2), jnp.int32)
    ei = edge_indices_list.astype(jnp.int32).reshape(-1)
    aggs = _sc_aggregate(h, ei)
    out_pad = _final_sum(y[0], aggs)
    return out_pad[:N_NODES_C]
